# baseline (speedup 1.0000x reference)
import functools

import numpy as np

import jax
import jax.numpy as jnp
from jax import lax
from jax.experimental import pallas as pl
from jax.experimental.pallas import tpu as pltpu
from jax.experimental.pallas import tpu_sc as plsc

D = 64
L = 16
NCH = D // L
N_TOTAL = 32768
NC, NS = 2, 16
NW = NC * NS
PER_W = N_TOTAL // NW
CHUNK = 256
NCHUNKS = PER_W // CHUNK
GPC = CHUNK // L
IDX_ROW = 128
ROWS_PER_CHUNK = CHUNK // IDX_ROW

_GDN = lax.GatherDimensionNumbers(
    offset_dims=(), collapsed_slice_dims=(0,), start_index_map=(0,))


def _lane_perm(v, perm):
    return lax.gather(v, perm, _GDN, slice_sizes=(1,),
                      mode=lax.GatherScatterMode.PROMISE_IN_BOUNDS)


def _body(z_hbm, idxi_hbm, idxj_hbm, w2t_hbm, w3_hbm, out_hbm,
          idxi_v, idxj_v, zi0, zj0, zi1, zj1, out_v, w2t_v, w3_v, sem):
    wid = lax.axis_index("s") * NC + lax.axis_index("c")

    iota = lax.iota(jnp.int32, L)
    perms = [(iota ^ (1 << k)).reshape(L, 1) for k in range(4)]
    masks = [iota == r for r in range(L)]

    irow0 = wid * (PER_W // IDX_ROW)
    pltpu.sync_copy(idxi_hbm.at[pl.ds(irow0, PER_W // IDX_ROW)], idxi_v)
    pltpu.sync_copy(idxj_hbm.at[pl.ds(irow0, PER_W // IDX_ROW)], idxj_v)

    zbufs = [(zi0, zj0), (zi1, zj1)]

    def fire(c):
        zi, zj = zbufs[c % 2]
        hs = []
        for k in range(ROWS_PER_CHUNK):
            r = c * ROWS_PER_CHUNK + k
            hs.append(pltpu.async_copy(
                z_hbm.at[idxi_v.at[r]], zi.at[pl.ds(k * IDX_ROW, IDX_ROW)], sem))
            hs.append(pltpu.async_copy(
                z_hbm.at[idxj_v.at[r]], zj.at[pl.ds(k * IDX_ROW, IDX_ROW)], sem))
        return hs

    pending = fire(0)

    pltpu.sync_copy(w2t_hbm, w2t_v)
    pltpu.sync_copy(w3_hbm, w3_v)
    u_chunks = None
    for ccv in range(NCH):
        w3a_vec = w3_v[pl.ds(ccv * L, L)]
        for e in range(L):
            s = w3a_vec[e]
            cc = ccv * L + e
            if u_chunks is None:
                u_chunks = [w2t_v[cc, pl.ds(kc * L, L)] * s
                            for kc in range(2 * NCH)]
            else:
                for kc in range(2 * NCH):
                    u_chunks[kc] = (u_chunks[kc]
                                    + w2t_v[cc, pl.ds(kc * L, L)] * s)
    ua = u_chunks[:NCH]
    ub = u_chunks[NCH:]
    w3b = [w3_v[pl.ds(D + cc * L, L)] for cc in range(NCH)]

    one = jnp.float32(1.0)
    zero = jnp.float32(0.0)

    for c in range(NCHUNKS):
        for h in pending:
            h.wait()
        if c + 1 < NCHUNKS:
            pending = fire(c + 1)
        zi, zj = zbufs[c % 2]

        def group(g, carry, zi=zi, zj=zj, c=c):
            sums = []
            for r in range(L):
                row = g * L + r
                acc = None
                for cc in range(NCH):
                    a = zi[row, pl.ds(cc * L, L)]
                    b = zj[row, pl.ds(cc * L, L)]
                    t = (jnp.maximum(a, zero) * ua[cc]
                         + jnp.maximum(b, zero) * ub[cc]
                         + (a * b) * w3b[cc])
                    acc = t if acc is None else acc + t
                for p in perms:
                    acc = acc + _lane_perm(acc, p)
                sums.append(jnp.where(masks[r], acc, zero))
            while len(sums) > 1:
                sums = [sums[i] + sums[i + 1] for i in range(0, len(sums), 2)]
            tot = sums[0]
            sig = one / (one + jnp.exp(-tot))
            out_v[pl.ds(c * CHUNK + g * L, L)] = sig
            return carry

        lax.fori_loop(0, GPC, group, jnp.int32(0))

    pltpu.sync_copy(out_v, out_hbm.at[pl.ds(wid * PER_W, PER_W)])


@jax.jit
def _run(z, idxi, idxj, w2t, w3flat):
    mesh = plsc.VectorSubcoreMesh(core_axis_name="c", subcore_axis_name="s")
    k = functools.partial(
        pl.kernel,
        mesh=mesh,
        compiler_params=pltpu.CompilerParams(use_tc_tiling_on_sc=False),
        out_type=jax.ShapeDtypeStruct((N_TOTAL,), jnp.float32),
        scratch_types=[
            pltpu.VMEM((PER_W // IDX_ROW, IDX_ROW), jnp.int32),
            pltpu.VMEM((PER_W // IDX_ROW, IDX_ROW), jnp.int32),
            pltpu.VMEM((CHUNK, D), jnp.float32),
            pltpu.VMEM((CHUNK, D), jnp.float32),
            pltpu.VMEM((CHUNK, D), jnp.float32),
            pltpu.VMEM((CHUNK, D), jnp.float32),
            pltpu.VMEM((PER_W,), jnp.float32),
            pltpu.VMEM((D, 2 * D), jnp.float32),
            pltpu.VMEM((2 * D,), jnp.float32),
            pltpu.SemaphoreType.DMA,
        ],
    )(_body)
    return k(z, idxi, idxj, w2t, w3flat)


def kernel(X, train_edges, train_false_edges, z, weight_two, weight_three):
    edges = jnp.concatenate([train_edges, train_false_edges], axis=0)
    idxi = edges[:, 0].reshape(N_TOTAL // IDX_ROW, IDX_ROW)
    idxj = edges[:, 1].reshape(N_TOTAL // IDX_ROW, IDX_ROW)
    out = _run(z, idxi, idxj, weight_two.T, weight_three.reshape(2 * D))
    return out.reshape(N_TOTAL, 1)
